# baseline (speedup 1.0000x reference)
import functools

import jax
import jax.numpy as jnp
import numpy as np
from jax import lax
from jax.experimental import pallas as pl
from jax.experimental.pallas import tpu as pltpu
from jax.experimental.pallas import tpu_sc as plsc

_LEVELS = [8, 8, 8, 5, 5, 5]
_NQ = 8
_EPS = 1e-3

_F = np.float32
_lev = np.array(_LEVELS, np.float32)
_hl = ((_lev - _F(1)) * _F(1.0 + _EPS) / _F(2)).astype(np.float32)
_off = np.where(_lev % 2 == 0, _F(0.5), _F(0.0)).astype(np.float32)
_shift = np.arctanh(_off / _hl).astype(np.float32)
_hw = np.floor(_lev / 2).astype(np.float32)
_basis = np.array([1, 8, 64, 512, 2560, 12800], np.float32)
_A = (_F(2) * _hl).astype(np.float32)
_B = (-(_hl + _off)).astype(np.float32)
_b0 = (_F(-2) * _shift).astype(np.float32)
_MAGIC = float(_F(12582912.0))
_K0 = float(_F(np.sum(_hw.astype(np.float64) * _basis.astype(np.float64))))
_AI = [(_F(-2) * np.power(_lev - _F(1), _F(i))).astype(np.float32)
       for i in range(_NQ)]
_QS = [(np.power(_lev - _F(1), _F(-i)) / _hw).astype(np.float32)
       for i in range(_NQ)]

_lev8 = np.array(_LEVELS + [2, 2], np.float32)
_hl8 = ((_lev8 - _F(1)) * _F(1.0 + _EPS) / _F(2)).astype(np.float32)
_off8 = np.where(_lev8 % 2 == 0, _F(0.5), _F(0.0)).astype(np.float32)
_shift8 = np.arctanh(_off8 / _hl8).astype(np.float32)
_C8 = np.stack([_hl8, _off8, _shift8], axis=1)

_NC = 2
_NS = 16
_LANES = 16


def _zin_body(x_ref, win_ref, bin_ref, c_ref, rT_ref):
    hl = c_ref[:, 0:1]
    off = c_ref[:, 1:2]
    shift = c_ref[:, 2:3]
    z = jax.lax.dot_general(
        x_ref[...], win_ref[...], (((1,), (0,)), ((), ())),
        preferred_element_type=jnp.float32)
    zT = z.T + bin_ref[...]
    rT_ref[...] = jnp.tanh(zT + shift) * hl - off


def _sc_fsq_body(TW, rT_hbm, qT_hbm, idxT_hbm, z_v, q_v, idx_v):
    wid = lax.axis_index("s") * _NC + lax.axis_index("c")
    base = wid * TW
    for c in range(6):
        pltpu.sync_copy(rT_hbm.at[c, pl.ds(base, TW)], z_v.at[c])

    def one(sl):
        r0 = [z_v[c, sl] for c in range(6)]
        r = list(r0)
        for i in range(_NQ):
            acc = None
            for c in range(6):
                e = jnp.exp(r[c] * _AI[i][c] + _b0[c])
                zb = _A[c] / (e + _F(1.0)) + _B[c]
                rnd = (zb + _F(_MAGIC)) - _F(_MAGIC)
                term = rnd if _basis[c] == 1.0 else rnd * _basis[c]
                acc = term if acc is None else acc + term
                r[c] = r[c] - rnd * _QS[i][c]
            idx_v[i, sl] = (acc + _F(_K0)).astype(jnp.int32)
        for c in range(6):
            q_v[c, sl] = r0[c] - r[c]

    def chunk(t, carry):
        for u in range(4):
            one(pl.ds(t * (4 * _LANES) + u * _LANES, _LANES))
        return carry

    lax.fori_loop(0, TW // (4 * _LANES), chunk, 0)

    for c in range(6):
        pltpu.sync_copy(q_v.at[c], qT_hbm.at[c, pl.ds(base, TW)])
    for i in range(_NQ):
        pltpu.sync_copy(idx_v.at[i], idxT_hbm.at[i, pl.ds(base, TW)])


def _proj_out_body(qT_ref, wout_ref, out_ref):
    q6 = qT_ref[...]
    blk = q6.shape[1]
    q8 = jnp.concatenate(
        [q6, jnp.ones((1, blk), jnp.float32),
         jnp.zeros((1, blk), jnp.float32)], axis=0)
    out_ref[...] = jax.lax.dot_general(
        q8, wout_ref[...], (((0,), (0,)), ((), ())),
        preferred_element_type=jnp.float32)


def _proj_out_body2(prev_ref, qT_ref, wout_ref, out_ref):
    del prev_ref
    _proj_out_body(qT_ref, wout_ref, out_ref)


_CH = 1


def kernel(x, W_in, b_in, W_out, b_out):
    B, N, D = x.shape
    T = B * N
    x2 = x.reshape(T, D)
    win8 = jnp.zeros((D, 8), jnp.float32).at[:, :6].set(W_in)
    bin8 = jnp.zeros((8, 1), jnp.float32).at[:6, 0].set(b_in)
    wout8 = jnp.zeros((8, D), jnp.float32).at[:6, :].set(W_out).at[6, :].set(b_out)
    c8 = jnp.asarray(_C8)

    BLK = 2048
    TC = T // _CH
    nblk = TC // BLK
    TW = TC // (_NC * _NS)
    mesh = plsc.VectorSubcoreMesh(core_axis_name="c", subcore_axis_name="s")
    sc_fn = functools.partial(
        pl.kernel,
        mesh=mesh,
        out_type=[
            jax.ShapeDtypeStruct((6, TC), jnp.float32),
            jax.ShapeDtypeStruct((8, TC), jnp.int32),
        ],
        scratch_types=[
            pltpu.VMEM((6, TW), jnp.float32),
            pltpu.VMEM((6, TW), jnp.float32),
            pltpu.VMEM((8, TW), jnp.int32),
        ],
    )(functools.partial(_sc_fsq_body, TW))

    qts, idxs = [], []
    for h in range(_CH):
        rT = pl.pallas_call(
            _zin_body,
            grid=(nblk,),
            in_specs=[
                pl.BlockSpec((BLK, D), lambda i, h=h: (h * nblk + i, 0)),
                pl.BlockSpec((D, 8), lambda i: (0, 0)),
                pl.BlockSpec((8, 1), lambda i: (0, 0)),
                pl.BlockSpec((8, 3), lambda i: (0, 0)),
            ],
            out_specs=pl.BlockSpec((8, BLK), lambda i: (0, i)),
            out_shape=jax.ShapeDtypeStruct((8, TC), jnp.float32),
        )(x2, win8, bin8, c8)
        qT, idxT = sc_fn(rT)
        qts.append(qT)
        idxs.append(idxT)

    out = pl.pallas_call(
        _proj_out_body,
        grid=(nblk,),
        in_specs=[
            pl.BlockSpec((6, BLK), lambda i: (0, i)),
            pl.BlockSpec((8, D), lambda i: (0, 0)),
        ],
        out_specs=pl.BlockSpec((BLK, D), lambda i: (i, 0)),
        out_shape=jax.ShapeDtypeStruct((T, D), jnp.float32),
    )(qts[0], wout8)
    for h in range(1, _CH):
        out = pl.pallas_call(
            _proj_out_body2,
            grid=(nblk,),
            in_specs=[
                pl.BlockSpec(memory_space=pl.ANY),
                pl.BlockSpec((6, BLK), lambda i: (0, i)),
                pl.BlockSpec((8, D), lambda i: (0, 0)),
            ],
            out_specs=pl.BlockSpec((BLK, D), lambda i, h=h: (h * nblk + i, 0)),
            out_shape=jax.ShapeDtypeStruct((T, D), jnp.float32),
            input_output_aliases={0: 0},
        )(out, qts[h], wout8)

    idxT = jnp.concatenate(idxs, axis=1)
    return out.reshape(B, N, D), idxT.T.reshape(B, N, _NQ)


# speedup vs baseline: 1.1894x; 1.1894x over previous
import functools

import jax
import jax.numpy as jnp
import numpy as np
from jax import lax
from jax.experimental import pallas as pl
from jax.experimental.pallas import tpu as pltpu
from jax.experimental.pallas import tpu_sc as plsc

_LEVELS = [8, 8, 8, 5, 5, 5]
_NQ = 8
_EPS = 1e-3

_F = np.float32
_lev = np.array(_LEVELS, np.float32)
_hl = ((_lev - _F(1)) * _F(1.0 + _EPS) / _F(2)).astype(np.float32)
_off = np.where(_lev % 2 == 0, _F(0.5), _F(0.0)).astype(np.float32)
_shift = np.arctanh(_off / _hl).astype(np.float32)
_hw = np.floor(_lev / 2).astype(np.float32)
_basis = np.array([1, 8, 64, 512, 2560, 12800], np.float32)
_A = (_F(2) * _hl).astype(np.float32)
_B = (-(_hl + _off)).astype(np.float32)
_b0 = (_F(-2) * _shift).astype(np.float32)
_MAGIC = float(_F(12582912.0))
_K0 = float(_F(np.sum(_hw.astype(np.float64) * _basis.astype(np.float64))))
_AI = [(_F(-2) * np.power(_lev - _F(1), _F(i))).astype(np.float32)
       for i in range(_NQ)]
_QS = [(np.power(_lev - _F(1), _F(-i)) / _hw).astype(np.float32)
       for i in range(_NQ)]

_lev8 = np.array(_LEVELS + [2, 2], np.float32)
_hl8 = ((_lev8 - _F(1)) * _F(1.0 + _EPS) / _F(2)).astype(np.float32)
_off8 = np.where(_lev8 % 2 == 0, _F(0.5), _F(0.0)).astype(np.float32)
_shift8 = np.arctanh(_off8 / _hl8).astype(np.float32)
_C8 = np.stack([_hl8, _off8, _shift8], axis=1)

_NC = 2
_NS = 16
_LANES = 16


def _zin_body(x_ref, win_ref, bin_ref, c_ref, rT_ref):
    hl = c_ref[:, 0:1]
    off = c_ref[:, 1:2]
    shift = c_ref[:, 2:3]
    z = jax.lax.dot_general(
        x_ref[...], win_ref[...], (((1,), (0,)), ((), ())),
        preferred_element_type=jnp.float32)
    zT = z.T + bin_ref[...]
    rT_ref[...] = jnp.tanh(zT + shift) * hl - off


def _sc_fsq_body(TW, rT_hbm, qT_hbm, idxT_hbm, z_v, q_v, idx_v):
    wid = lax.axis_index("s") * _NC + lax.axis_index("c")
    base = wid * TW
    for c in range(6):
        pltpu.sync_copy(rT_hbm.at[c, pl.ds(base, TW)], z_v.at[c])

    def one(sl):
        r0 = [z_v[c, sl] for c in range(6)]
        r = list(r0)
        for i in range(_NQ):
            acc = None
            for c in range(6):
                e = jnp.exp(r[c] * _AI[i][c] + _b0[c])
                zb = _A[c] / (e + _F(1.0)) + _B[c]
                rnd = (zb + _F(_MAGIC)) - _F(_MAGIC)
                term = rnd if _basis[c] == 1.0 else rnd * _basis[c]
                acc = term if acc is None else acc + term
                r[c] = r[c] - rnd * _QS[i][c]
            idx_v[i, sl] = (acc + _F(_K0)).astype(jnp.int32)
        for c in range(6):
            q_v[c, sl] = r0[c] - r[c]

    def chunk(t, carry):
        one(pl.ds(t * (2 * _LANES), _LANES))
        one(pl.ds(t * (2 * _LANES) + _LANES, _LANES))
        return carry

    lax.fori_loop(0, TW // (2 * _LANES), chunk, 0)

    for c in range(6):
        pltpu.sync_copy(q_v.at[c], qT_hbm.at[c, pl.ds(base, TW)])
    for i in range(_NQ):
        pltpu.sync_copy(idx_v.at[i], idxT_hbm.at[i, pl.ds(base, TW)])


def _proj_out_body(qT_ref, wout_ref, out_ref):
    q6 = qT_ref[...]
    blk = q6.shape[1]
    q8 = jnp.concatenate(
        [q6, jnp.ones((1, blk), jnp.float32),
         jnp.zeros((1, blk), jnp.float32)], axis=0)
    out_ref[...] = jax.lax.dot_general(
        q8, wout_ref[...], (((0,), (0,)), ((), ())),
        preferred_element_type=jnp.float32)


def _proj_out_body2(prev_ref, qT_ref, wout_ref, out_ref):
    del prev_ref
    _proj_out_body(qT_ref, wout_ref, out_ref)


_CH = 1


def kernel(x, W_in, b_in, W_out, b_out):
    B, N, D = x.shape
    T = B * N
    x2 = x.reshape(T, D)
    win8 = jnp.zeros((D, 8), jnp.float32).at[:, :6].set(W_in)
    bin8 = jnp.zeros((8, 1), jnp.float32).at[:6, 0].set(b_in)
    wout8 = jnp.zeros((8, D), jnp.float32).at[:6, :].set(W_out).at[6, :].set(b_out)
    c8 = jnp.asarray(_C8)

    BLK = 2048
    TC = T // _CH
    nblk = TC // BLK
    TW = TC // (_NC * _NS)
    mesh = plsc.VectorSubcoreMesh(core_axis_name="c", subcore_axis_name="s")
    sc_fn = functools.partial(
        pl.kernel,
        mesh=mesh,
        out_type=[
            jax.ShapeDtypeStruct((6, TC), jnp.float32),
            jax.ShapeDtypeStruct((8, TC), jnp.int32),
        ],
        scratch_types=[
            pltpu.VMEM((6, TW), jnp.float32),
            pltpu.VMEM((6, TW), jnp.float32),
            pltpu.VMEM((8, TW), jnp.int32),
        ],
    )(functools.partial(_sc_fsq_body, TW))

    qts, idxs = [], []
    for h in range(_CH):
        rT = pl.pallas_call(
            _zin_body,
            grid=(nblk,),
            in_specs=[
                pl.BlockSpec((BLK, D), lambda i, h=h: (h * nblk + i, 0)),
                pl.BlockSpec((D, 8), lambda i: (0, 0)),
                pl.BlockSpec((8, 1), lambda i: (0, 0)),
                pl.BlockSpec((8, 3), lambda i: (0, 0)),
            ],
            out_specs=pl.BlockSpec((8, BLK), lambda i: (0, i)),
            out_shape=jax.ShapeDtypeStruct((8, TC), jnp.float32),
        )(x2, win8, bin8, c8)
        qT, idxT = sc_fn(rT)
        qts.append(qT)
        idxs.append(idxT)

    out = pl.pallas_call(
        _proj_out_body,
        grid=(nblk,),
        in_specs=[
            pl.BlockSpec((6, BLK), lambda i: (0, i)),
            pl.BlockSpec((8, D), lambda i: (0, 0)),
        ],
        out_specs=pl.BlockSpec((BLK, D), lambda i: (i, 0)),
        out_shape=jax.ShapeDtypeStruct((T, D), jnp.float32),
    )(qts[0], wout8)
    for h in range(1, _CH):
        out = pl.pallas_call(
            _proj_out_body2,
            grid=(nblk,),
            in_specs=[
                pl.BlockSpec(memory_space=pl.ANY),
                pl.BlockSpec((6, BLK), lambda i: (0, i)),
                pl.BlockSpec((8, D), lambda i: (0, 0)),
            ],
            out_specs=pl.BlockSpec((BLK, D), lambda i, h=h: (h * nblk + i, 0)),
            out_shape=jax.ShapeDtypeStruct((T, D), jnp.float32),
            input_output_aliases={0: 0},
        )(out, qts[h], wout8)

    idxT = jnp.concatenate(idxs, axis=1)
    return out.reshape(B, N, D), idxT.T.reshape(B, N, _NQ)


# speedup vs baseline: 1.4534x; 1.2219x over previous
import functools

import jax
import jax.numpy as jnp
import numpy as np
from jax import lax
from jax.experimental import pallas as pl
from jax.experimental.pallas import tpu as pltpu
from jax.experimental.pallas import tpu_sc as plsc

_LEVELS = [8, 8, 8, 5, 5, 5]
_NQ = 8
_EPS = 1e-3

_F = np.float32
_lev = np.array(_LEVELS, np.float32)
_hl = ((_lev - _F(1)) * _F(1.0 + _EPS) / _F(2)).astype(np.float32)
_off = np.where(_lev % 2 == 0, _F(0.5), _F(0.0)).astype(np.float32)
_shift = np.arctanh(_off / _hl).astype(np.float32)
_hw = np.floor(_lev / 2).astype(np.float32)
_basis = np.array([1, 8, 64, 512, 2560, 12800], np.float32)
_A = (_F(2) * _hl).astype(np.float32)
_B = (-(_hl + _off)).astype(np.float32)
_b0 = (_F(-2) * _shift).astype(np.float32)
_MAGIC = float(_F(12582912.0))
_K0 = float(_F(np.sum(_hw.astype(np.float64) * _basis.astype(np.float64))))
_AI = [(_F(-2) * np.power(_lev - _F(1), _F(i))).astype(np.float32)
       for i in range(_NQ)]
_QS = [(np.power(_lev - _F(1), _F(-i)) / _hw).astype(np.float32)
       for i in range(_NQ)]

_lev8 = np.array(_LEVELS + [2, 2], np.float32)
_hl8 = ((_lev8 - _F(1)) * _F(1.0 + _EPS) / _F(2)).astype(np.float32)
_off8 = np.where(_lev8 % 2 == 0, _F(0.5), _F(0.0)).astype(np.float32)
_shift8 = np.arctanh(_off8 / _hl8).astype(np.float32)
_C8 = np.stack([_hl8, _off8, _shift8], axis=1)

_NC = 2
_NS = 16
_LANES = 16


def _fsq_fused_body(x_ref, win_ref, bin_ref, wout_ref, c_ref,
                    out_ref, idx_ref, q_scr):
    g = pl.program_id(0)
    hl = c_ref[:, 0:1]
    off = c_ref[:, 1:2]
    shift = c_ref[:, 2:3]
    basis = c_ref[:, 4:5]
    k0 = c_ref[0:1, 21:22]
    row = jax.lax.broadcasted_iota(jnp.int32, (8, 1), 0)

    @pl.when(g > 0)
    def _drain():
        out_ref[...] = jax.lax.dot_general(
            q_scr[...], wout_ref[...], (((0,), (0,)), ((), ())),
            preferred_element_type=jnp.float32)

    z = jax.lax.dot_general(
        x_ref[...], win_ref[...], (((1,), (0,)), ((), ())),
        preferred_element_type=jnp.float32)
    zT = z.T + bin_ref[...]

    r = jnp.tanh(zT + shift) * hl - off
    q = jnp.zeros_like(r)
    idx_rows = []
    for i in range(_NQ):
        inv = c_ref[:, 5 + i:6 + i]
        qs = c_ref[:, 13 + i:14 + i]
        zb = jnp.tanh(r * inv + shift) * hl - off
        rnd = jnp.round(zb)
        idxf = jnp.sum(rnd * basis, axis=0) + k0[0]
        idx_rows.append(idxf.astype(jnp.int32))
        quant = rnd * qs
        r = r - quant
        q = q + quant

    q_scr[...] = jnp.where(row == 6, 1.0, q)
    idx_ref[...] = jnp.stack(idx_rows, axis=0)


def _zin_body(x_ref, win_ref, bin_ref, c_ref, rT_ref):
    hl = c_ref[:, 0:1]
    off = c_ref[:, 1:2]
    shift = c_ref[:, 2:3]
    z = jax.lax.dot_general(
        x_ref[...], win_ref[...], (((1,), (0,)), ((), ())),
        preferred_element_type=jnp.float32)
    zT = z.T + bin_ref[...]
    rT_ref[...] = jnp.tanh(zT + shift) * hl - off


def _sc_fsq_body(TW, rT_hbm, qT_hbm, idxT_hbm, z_v, q_v, idx_v):
    wid = lax.axis_index("s") * _NC + lax.axis_index("c")
    base = wid * TW
    for c in range(6):
        pltpu.sync_copy(rT_hbm.at[c, pl.ds(base, TW)], z_v.at[c])

    def one(sl):
        r0 = [z_v[c, sl] for c in range(6)]
        r = list(r0)
        for i in range(_NQ):
            acc = None
            for c in range(6):
                e = jnp.exp(r[c] * _AI[i][c] + _b0[c])
                zb = _A[c] / (e + _F(1.0)) + _B[c]
                rnd = (zb + _F(_MAGIC)) - _F(_MAGIC)
                term = rnd if _basis[c] == 1.0 else rnd * _basis[c]
                acc = term if acc is None else acc + term
                r[c] = r[c] - rnd * _QS[i][c]
            idx_v[i, sl] = (acc + _F(_K0)).astype(jnp.int32)
        for c in range(6):
            q_v[c, sl] = r0[c] - r[c]

    def chunk(t, carry):
        one(pl.ds(t * (2 * _LANES), _LANES))
        one(pl.ds(t * (2 * _LANES) + _LANES, _LANES))
        return carry

    lax.fori_loop(0, TW // (2 * _LANES), chunk, 0)

    for c in range(6):
        pltpu.sync_copy(q_v.at[c], qT_hbm.at[c, pl.ds(base, TW)])
    for i in range(_NQ):
        pltpu.sync_copy(idx_v.at[i], idxT_hbm.at[i, pl.ds(base, TW)])


def _proj_out_body(qT_ref, wout_ref, out_ref):
    q6 = qT_ref[...]
    blk = q6.shape[1]
    q8 = jnp.concatenate(
        [q6, jnp.ones((1, blk), jnp.float32),
         jnp.zeros((1, blk), jnp.float32)], axis=0)
    out_ref[...] = jax.lax.dot_general(
        q8, wout_ref[...], (((0,), (0,)), ((), ())),
        preferred_element_type=jnp.float32)


def _proj_out_body2(prev_ref, qT_ref, wout_ref, out_ref):
    del prev_ref
    _proj_out_body(qT_ref, wout_ref, out_ref)


_SC_BLKS = 2


def kernel(x, W_in, b_in, W_out, b_out):
    B, N, D = x.shape
    T = B * N
    x2 = x.reshape(T, D)
    win8 = jnp.zeros((D, 8), jnp.float32).at[:, :6].set(W_in)
    bin8 = jnp.zeros((8, 1), jnp.float32).at[:6, 0].set(b_in)
    wout8 = jnp.zeros((8, D), jnp.float32).at[:6, :].set(W_out).at[6, :].set(b_out)
    c8 = jnp.asarray(_C8)

    lev = jnp.array(_LEVELS + [2, 2], dtype=jnp.float32)
    half_l = (lev - 1.0) * (1.0 + _EPS) / 2.0
    offset = jnp.where(jnp.mod(lev, 2.0) == 0.0, 0.5, 0.0)
    shift = jnp.arctanh(offset / half_l)
    hw = jnp.floor(lev / 2.0)
    basis = jnp.concatenate([
        jnp.array(np.concatenate(([1], np.cumprod(_LEVELS[:-1]))),
                  dtype=jnp.float32),
        jnp.zeros((2,), jnp.float32)])
    invs = [(lev - 1.0) ** float(i) for i in range(_NQ)]
    qss = [((lev - 1.0) ** (-float(i))) / hw for i in range(_NQ)]
    k0col = jnp.full((8,), jnp.sum(hw * basis), jnp.float32)
    cols = [half_l, offset, shift, hw, basis] + invs + qss + [k0col]
    cols += [jnp.zeros((8,), jnp.float32)] * (24 - len(cols))
    consts = jnp.stack(cols, axis=1)

    BLK = 2048
    nblk = T // BLK
    ntc = nblk - _SC_BLKS
    TSC = _SC_BLKS * BLK
    TW = TSC // (_NC * _NS)
    last_tc = ntc - 1

    out0, idxT0 = pl.pallas_call(
        _fsq_fused_body,
        grid=(ntc + 1,),
        in_specs=[
            pl.BlockSpec((BLK, D), lambda i: (jnp.minimum(i, last_tc), 0)),
            pl.BlockSpec((D, 8), lambda i: (0, 0)),
            pl.BlockSpec((8, 1), lambda i: (0, 0)),
            pl.BlockSpec((8, D), lambda i: (0, 0)),
            pl.BlockSpec((8, 24), lambda i: (0, 0)),
        ],
        out_specs=[
            pl.BlockSpec((BLK, D), lambda i: (jnp.maximum(i - 1, 0), 0)),
            pl.BlockSpec((8, BLK), lambda i: (0, jnp.minimum(i, last_tc))),
        ],
        out_shape=[
            jax.ShapeDtypeStruct((T, D), jnp.float32),
            jax.ShapeDtypeStruct((8, T), jnp.int32),
        ],
        scratch_shapes=[pltpu.VMEM((8, BLK), jnp.float32)],
    )(x2, win8, bin8, wout8, consts)

    rT = pl.pallas_call(
        _zin_body,
        grid=(_SC_BLKS,),
        in_specs=[
            pl.BlockSpec((BLK, D), lambda i: (ntc + i, 0)),
            pl.BlockSpec((D, 8), lambda i: (0, 0)),
            pl.BlockSpec((8, 1), lambda i: (0, 0)),
            pl.BlockSpec((8, 3), lambda i: (0, 0)),
        ],
        out_specs=pl.BlockSpec((8, BLK), lambda i: (0, i)),
        out_shape=jax.ShapeDtypeStruct((8, TSC), jnp.float32),
    )(x2, win8, bin8, c8)

    mesh = plsc.VectorSubcoreMesh(core_axis_name="c", subcore_axis_name="s")
    sc_fn = functools.partial(
        pl.kernel,
        mesh=mesh,
        out_type=[
            jax.ShapeDtypeStruct((6, TSC), jnp.float32),
            jax.ShapeDtypeStruct((8, TSC), jnp.int32),
        ],
        scratch_types=[
            pltpu.VMEM((6, TW), jnp.float32),
            pltpu.VMEM((6, TW), jnp.float32),
            pltpu.VMEM((8, TW), jnp.int32),
        ],
    )(functools.partial(_sc_fsq_body, TW))
    qT, idxT1 = sc_fn(rT)

    out = pl.pallas_call(
        _proj_out_body2,
        grid=(_SC_BLKS,),
        in_specs=[
            pl.BlockSpec(memory_space=pl.ANY),
            pl.BlockSpec((6, BLK), lambda i: (0, i)),
            pl.BlockSpec((8, D), lambda i: (0, 0)),
        ],
        out_specs=pl.BlockSpec((BLK, D), lambda i: (ntc + i, 0)),
        out_shape=jax.ShapeDtypeStruct((T, D), jnp.float32),
        input_output_aliases={0: 0},
    )(out0, qT, wout8)

    idxT = jnp.concatenate([idxT0[:, :ntc * BLK], idxT1], axis=1)
    return out.reshape(B, N, D), idxT.T.reshape(B, N, _NQ)
